# Initial kernel scaffold; baseline (speedup 1.0000x reference)
import jax
import jax.numpy as jnp
from jax.experimental import pallas as pl


def kernel(s_idx, r_idx, o_idx, ent_weight, rel_weight):
    raise NotImplementedError("write your pallas kernel here")


# speedup vs baseline: 7.4560x; 7.4560x over previous
import functools

import numpy as np
import jax
import jax.numpy as jnp
from jax import lax
from jax.experimental import pallas as pl
from jax.experimental.pallas import tpu as pltpu
from jax.experimental.pallas import tpu_sc as plsc

N_NODES = 100000
N_RELS = 1000
EMB = 128
B = 16384

NC = 2
NS = 16
NW = NC * NS
BPW = B // NW
CHUNK = 64
NCHUNK = BPW // CHUNK
ROW = 2 * EMB
L = 16

_MAGIC = np.int32(0x5F3759DF)


def _rsqrt(x):
    i = plsc.bitcast(x, jnp.int32)
    y = plsc.bitcast(_MAGIC - (i >> 1), jnp.float32)
    for _ in range(2):
        y = y * (1.5 - 0.5 * x * y * y)
    return y


def _body(s_hbm, r_hbm, o_hbm, ent_hbm, rel_hbm, out_hbm,
          sidx_v, ridx_v, oidx_v, out_v,
          bs0, bo0, br0, bs1, bo1, br1, sem0, sem1):
    cid = lax.axis_index("c")
    sid = lax.axis_index("s")
    wid = sid * NC + cid
    base = wid * BPW

    pltpu.sync_copy(s_hbm.at[pl.ds(base, BPW)], sidx_v)
    pltpu.sync_copy(r_hbm.at[pl.ds(base, BPW)], ridx_v)
    pltpu.sync_copy(o_hbm.at[pl.ds(base, BPW)], oidx_v)

    bufs = [(bs0, bo0, br0, sem0), (bs1, bo1, br1, sem1)]

    def start(c):
        bs, bo, br, sem = bufs[c % 2]
        off = c * CHUNK
        hs = pltpu.async_copy(ent_hbm.at[sidx_v.at[pl.ds(off, CHUNK)]], bs, sem)
        ho = pltpu.async_copy(ent_hbm.at[oidx_v.at[pl.ds(off, CHUNK)]], bo, sem)
        hr = pltpu.async_copy(rel_hbm.at[ridx_v.at[pl.ds(off, CHUNK)]], br, sem)
        return (hs, ho, hr)

    iota = lax.iota(jnp.int32, L)
    mask0 = iota == 0
    idx_re = [iota * 2 + 2 * L * j for j in range(EMB // L)]
    idx_im = [iota * 2 + (2 * L * j + 1) for j in range(EMB // L)]

    pending = {0: start(0)}
    for c in range(NCHUNK):
        for h in pending.pop(c):
            h.wait()
        if c + 1 < NCHUNK:
            pending[c + 1] = start(c + 1)
        bs, bo, br, _ = bufs[c % 2]
        out_off = c * CHUNK

        def row_body(rr, carry, bs=bs, bo=bo, br=br, out_off=out_off):
            rs = jnp.full((L,), rr, jnp.int32)
            acc = jnp.zeros((L,), jnp.float32)
            for j in range(EMB // L):
                sre = plsc.load_gather(bs, [rs, idx_re[j]])
                sim = plsc.load_gather(bs, [rs, idx_im[j]])
                rre = plsc.load_gather(br, [rs, idx_re[j]])
                rim = plsc.load_gather(br, [rs, idx_im[j]])
                ore = plsc.load_gather(bo, [rs, idx_re[j]])
                oim = plsc.load_gather(bo, [rs, idx_im[j]])
                pre = sre * rre - sim * rim
                pim = sre * rim + sim * rre
                ip = jnp.minimum(_rsqrt(pre * pre + pim * pim), 1e18)
                io = jnp.minimum(_rsqrt(ore * ore + oim * oim), 1e9)
                dre = pre * ip - ore * io
                dim_ = pim * ip - oim * io
                acc = acc + dre * dre + dim_ * dim_
            tot = jnp.sum(acc)
            plsc.store_scatter(out_v, [rs + out_off],
                               jnp.full((L,), tot, jnp.float32), mask=mask0)
            return carry

        lax.fori_loop(0, CHUNK, row_body, 0)

    for k in range(BPW // L):
        x = out_v[pl.ds(k * L, L)]
        out_v[pl.ds(k * L, L)] = x * _rsqrt(x)

    pltpu.sync_copy(out_v, out_hbm.at[pl.ds(base, BPW)])


@jax.jit
def _run(s_idx, r_idx, o_idx, ent_weight, rel_weight):
    mesh = plsc.VectorSubcoreMesh(core_axis_name="c", subcore_axis_name="s")
    f = functools.partial(
        pl.kernel,
        mesh=mesh,
        compiler_params=pltpu.CompilerParams(use_tc_tiling_on_sc=False,
                                             needs_layout_passes=False),
        out_type=jax.ShapeDtypeStruct((B,), jnp.float32),
        scratch_types=[
            pltpu.VMEM((BPW,), jnp.int32),
            pltpu.VMEM((BPW,), jnp.int32),
            pltpu.VMEM((BPW,), jnp.int32),
            pltpu.VMEM((BPW,), jnp.float32),
            pltpu.VMEM((CHUNK, ROW), jnp.float32),
            pltpu.VMEM((CHUNK, ROW), jnp.float32),
            pltpu.VMEM((CHUNK, ROW), jnp.float32),
            pltpu.VMEM((CHUNK, ROW), jnp.float32),
            pltpu.VMEM((CHUNK, ROW), jnp.float32),
            pltpu.VMEM((CHUNK, ROW), jnp.float32),
            pltpu.SemaphoreType.DMA,
            pltpu.SemaphoreType.DMA,
        ],
    )(_body)
    return f(s_idx, r_idx, o_idx, ent_weight, rel_weight)


def kernel(s_idx, r_idx, o_idx, ent_weight, rel_weight):
    return _run(s_idx, r_idx, o_idx, ent_weight, rel_weight)


# speedup vs baseline: 13.6731x; 1.8338x over previous
import functools

import numpy as np
import jax
import jax.numpy as jnp
from jax import lax
from jax.experimental import pallas as pl
from jax.experimental.pallas import tpu as pltpu
from jax.experimental.pallas import tpu_sc as plsc

N_NODES = 100000
N_RELS = 1000
EMB = 128
B = 16384

NC = 2
NS = 16
NW = NC * NS
BPW = B // NW
CHUNK = 64
NCHUNK = BPW // CHUNK
ROW = 2 * EMB
L = 16

_MAGIC = np.int32(0x5F3759DF)


def _rsqrt(x):
    i = plsc.bitcast(x, jnp.int32)
    y = plsc.bitcast(_MAGIC - (i >> 1), jnp.float32)
    for _ in range(2):
        y = y * (1.5 - 0.5 * x * y * y)
    return y


def _body(s_hbm, r_hbm, o_hbm, ent_hbm, rel_hbm, out_hbm,
          sidx_v, ridx_v, oidx_v, out_v,
          bs0, bo0, br0, bs1, bo1, br1, sem0, sem1):
    cid = lax.axis_index("c")
    sid = lax.axis_index("s")
    wid = sid * NC + cid
    base = wid * BPW

    pltpu.sync_copy(s_hbm.at[pl.ds(base, BPW)], sidx_v)
    pltpu.sync_copy(r_hbm.at[pl.ds(base, BPW)], ridx_v)
    pltpu.sync_copy(o_hbm.at[pl.ds(base, BPW)], oidx_v)

    bufs = [(bs0, bo0, br0, sem0), (bs1, bo1, br1, sem1)]

    def start(c):
        bs, bo, br, sem = bufs[c % 2]
        off = c * CHUNK
        hs = pltpu.async_copy(ent_hbm.at[sidx_v.at[pl.ds(off, CHUNK)]], bs, sem)
        ho = pltpu.async_copy(ent_hbm.at[oidx_v.at[pl.ds(off, CHUNK)]], bo, sem)
        hr = pltpu.async_copy(rel_hbm.at[ridx_v.at[pl.ds(off, CHUNK)]], br, sem)
        return (hs, ho, hr)

    iota = lax.iota(jnp.int32, L)
    mask0 = iota == 0
    idx_re = [iota * 2 + 2 * L * j for j in range(EMB // L)]
    idx_im = [iota * 2 + (2 * L * j + 1) for j in range(EMB // L)]

    pending = {0: start(0)}
    for c in range(NCHUNK):
        for h in pending.pop(c):
            h.wait()
        if c + 1 < NCHUNK:
            pending[c + 1] = start(c + 1)
        bs, bo, br, _ = bufs[c % 2]
        out_off = c * CHUNK

        def row_body(rr, carry, bs=bs, bo=bo, br=br, out_off=out_off):
            rs = jnp.full((L,), rr, jnp.int32)
            acc = jnp.zeros((L,), jnp.float32)
            for j in range(EMB // L):
                sre = plsc.load_gather(bs, [rs, idx_re[j]])
                sim = plsc.load_gather(bs, [rs, idx_im[j]])
                rre = plsc.load_gather(br, [rs, idx_re[j]])
                rim = plsc.load_gather(br, [rs, idx_im[j]])
                ore = plsc.load_gather(bo, [rs, idx_re[j]])
                oim = plsc.load_gather(bo, [rs, idx_im[j]])
                pre = sre * rre - sim * rim
                pim = sre * rim + sim * rre
                ip = jnp.minimum(_rsqrt(pre * pre + pim * pim), 1e18)
                io = jnp.minimum(_rsqrt(ore * ore + oim * oim), 1e9)
                dre = pre * ip - ore * io
                dim_ = pim * ip - oim * io
                acc = acc + dre * dre + dim_ * dim_
            tot = jnp.sum(acc)
            plsc.store_scatter(out_v, [rs + out_off],
                               jnp.full((L,), tot, jnp.float32), mask=mask0)
            return carry

        lax.fori_loop(0, CHUNK, row_body, 0)

    for k in range(BPW // L):
        x = out_v[pl.ds(k * L, L)]
        out_v[pl.ds(k * L, L)] = x * _rsqrt(x)

    pltpu.sync_copy(out_v, out_hbm.at[pl.ds(base, BPW)])


@jax.jit
def _run(s_idx, r_idx, o_idx, ent_weight, rel_weight):
    mesh = plsc.VectorSubcoreMesh(core_axis_name="c", subcore_axis_name="s")
    f = functools.partial(
        pl.kernel,
        mesh=mesh,
        compiler_params=pltpu.CompilerParams(use_tc_tiling_on_sc=True,
                                             needs_layout_passes=False),
        out_type=jax.ShapeDtypeStruct((B,), jnp.float32),
        scratch_types=[
            pltpu.VMEM((BPW,), jnp.int32),
            pltpu.VMEM((BPW,), jnp.int32),
            pltpu.VMEM((BPW,), jnp.int32),
            pltpu.VMEM((BPW,), jnp.float32),
            pltpu.VMEM((CHUNK, ROW), jnp.float32),
            pltpu.VMEM((CHUNK, ROW), jnp.float32),
            pltpu.VMEM((CHUNK, ROW), jnp.float32),
            pltpu.VMEM((CHUNK, ROW), jnp.float32),
            pltpu.VMEM((CHUNK, ROW), jnp.float32),
            pltpu.VMEM((CHUNK, ROW), jnp.float32),
            pltpu.SemaphoreType.DMA,
            pltpu.SemaphoreType.DMA,
        ],
    )(_body)
    return f(s_idx, r_idx, o_idx, ent_weight, rel_weight)


def kernel(s_idx, r_idx, o_idx, ent_weight, rel_weight):
    return _run(s_idx, r_idx, o_idx, ent_weight, rel_weight)


# speedup vs baseline: 14.8693x; 1.0875x over previous
import functools

import numpy as np
import jax
import jax.numpy as jnp
from jax import lax
from jax.experimental import pallas as pl
from jax.experimental.pallas import tpu as pltpu
from jax.experimental.pallas import tpu_sc as plsc

N_NODES = 100000
N_RELS = 1000
EMB = 128
B = 16384

NC = 2
NS = 16
NW = NC * NS
BPW = B // NW
CHUNK = 64
NCHUNK = BPW // CHUNK
ROW = 2 * EMB
L = 16

_MAGIC = np.int32(0x5F3759DF)


def _rsqrt(x):
    i = plsc.bitcast(x, jnp.int32)
    y = plsc.bitcast(_MAGIC - (i >> 1), jnp.float32)
    for _ in range(2):
        y = y * (1.5 - 0.5 * x * y * y)
    return y


def _rsqrt3(x):
    i = plsc.bitcast(x, jnp.int32)
    y = plsc.bitcast(_MAGIC - (i >> 1), jnp.float32)
    w = x * y * y
    return y * (1.875 + w * (-1.25 + 0.375 * w))


def _body(s_hbm, r_hbm, o_hbm, ent_hbm, rel_hbm, out_hbm,
          sidx_v, ridx_v, oidx_v, out_v,
          bs0, bo0, br0, bs1, bo1, br1, sem0, sem1):
    cid = lax.axis_index("c")
    sid = lax.axis_index("s")
    wid = sid * NC + cid
    base = wid * BPW

    pltpu.sync_copy(s_hbm.at[pl.ds(base, BPW)], sidx_v)
    pltpu.sync_copy(r_hbm.at[pl.ds(base, BPW)], ridx_v)
    pltpu.sync_copy(o_hbm.at[pl.ds(base, BPW)], oidx_v)

    bufs = [(bs0, bo0, br0, sem0), (bs1, bo1, br1, sem1)]

    def start(c):
        bs, bo, br, sem = bufs[c % 2]
        off = c * CHUNK
        hs = pltpu.async_copy(ent_hbm.at[sidx_v.at[pl.ds(off, CHUNK)]], bs, sem)
        ho = pltpu.async_copy(ent_hbm.at[oidx_v.at[pl.ds(off, CHUNK)]], bo, sem)
        hr = pltpu.async_copy(rel_hbm.at[ridx_v.at[pl.ds(off, CHUNK)]], br, sem)
        return (hs, ho, hr)

    iota = lax.iota(jnp.int32, L)
    mask0 = iota == 0
    idx_re = [iota * 2 + 2 * L * j for j in range(EMB // L)]
    idx_im = [iota * 2 + (2 * L * j + 1) for j in range(EMB // L)]

    pending = {0: start(0)}
    for c in range(NCHUNK):
        for h in pending.pop(c):
            h.wait()
        if c + 1 < NCHUNK:
            pending[c + 1] = start(c + 1)
        bs, bo, br, _ = bufs[c % 2]
        out_off = c * CHUNK

        def one_row(rr, bs, bo, br):
            rs = jnp.full((L,), rr, jnp.int32)
            acc = jnp.zeros((L,), jnp.float32)
            for j in range(EMB // L):
                sre = plsc.load_gather(bs, [rs, idx_re[j]])
                sim = plsc.load_gather(bs, [rs, idx_im[j]])
                rre = plsc.load_gather(br, [rs, idx_re[j]])
                rim = plsc.load_gather(br, [rs, idx_im[j]])
                ore = plsc.load_gather(bo, [rs, idx_re[j]])
                oim = plsc.load_gather(bo, [rs, idx_im[j]])
                pre = sre * rre - sim * rim
                pim = sre * rim + sim * rre
                ip = jnp.minimum(_rsqrt3(pre * pre + pim * pim), 1e18)
                io = jnp.minimum(_rsqrt3(ore * ore + oim * oim), 1e9)
                dre = pre * ip - ore * io
                dim_ = pim * ip - oim * io
                acc = acc + dre * dre + dim_ * dim_
            tot = jnp.sum(acc)
            plsc.store_scatter(out_v, [rs + out_off],
                               jnp.full((L,), tot, jnp.float32), mask=mask0)

        def row_body(rp, carry, bs=bs, bo=bo, br=br):
            one_row(rp * 2, bs, bo, br)
            one_row(rp * 2 + 1, bs, bo, br)
            return carry

        lax.fori_loop(0, CHUNK // 2, row_body, 0)

    for k in range(BPW // L):
        x = out_v[pl.ds(k * L, L)]
        out_v[pl.ds(k * L, L)] = x * _rsqrt(x)

    pltpu.sync_copy(out_v, out_hbm.at[pl.ds(base, BPW)])


@jax.jit
def _run(s_idx, r_idx, o_idx, ent_weight, rel_weight):
    mesh = plsc.VectorSubcoreMesh(core_axis_name="c", subcore_axis_name="s")
    f = functools.partial(
        pl.kernel,
        mesh=mesh,
        compiler_params=pltpu.CompilerParams(use_tc_tiling_on_sc=True,
                                             needs_layout_passes=False),
        out_type=jax.ShapeDtypeStruct((B,), jnp.float32),
        scratch_types=[
            pltpu.VMEM((BPW,), jnp.int32),
            pltpu.VMEM((BPW,), jnp.int32),
            pltpu.VMEM((BPW,), jnp.int32),
            pltpu.VMEM((BPW,), jnp.float32),
            pltpu.VMEM((CHUNK, ROW), jnp.float32),
            pltpu.VMEM((CHUNK, ROW), jnp.float32),
            pltpu.VMEM((CHUNK, ROW), jnp.float32),
            pltpu.VMEM((CHUNK, ROW), jnp.float32),
            pltpu.VMEM((CHUNK, ROW), jnp.float32),
            pltpu.VMEM((CHUNK, ROW), jnp.float32),
            pltpu.SemaphoreType.DMA,
            pltpu.SemaphoreType.DMA,
        ],
    )(_body)
    return f(s_idx, r_idx, o_idx, ent_weight, rel_weight)


def kernel(s_idx, r_idx, o_idx, ent_weight, rel_weight):
    return _run(s_idx, r_idx, o_idx, ent_weight, rel_weight)


# speedup vs baseline: 16.5364x; 1.1121x over previous
import functools

import numpy as np
import jax
import jax.numpy as jnp
from jax import lax
from jax.experimental import pallas as pl
from jax.experimental.pallas import tpu as pltpu
from jax.experimental.pallas import tpu_sc as plsc

N_NODES = 100000
N_RELS = 1000
EMB = 128
B = 16384

NC = 2
NS = 16
NW = NC * NS
BPW = B // NW
CHUNK = 64
NCHUNK = BPW // CHUNK
ROW = 2 * EMB
L = 16

_MAGIC = np.int32(0x5F3759DF)


def _rsqrt(x):
    i = plsc.bitcast(x, jnp.int32)
    y = plsc.bitcast(_MAGIC - (i >> 1), jnp.float32)
    for _ in range(2):
        y = y * (1.5 - 0.5 * x * y * y)
    return y


def _rsqrt3(x):
    i = plsc.bitcast(x, jnp.int32)
    y = plsc.bitcast(_MAGIC - (i >> 1), jnp.float32)
    w = x * y * y
    return y * (1.875 + w * (-1.25 + 0.375 * w))


def _body(s_hbm, r_hbm, o_hbm, ent_hbm, rel_hbm, out_hbm,
          sidx_v, ridx_v, oidx_v, out_v,
          bs0, bo0, br0, bs1, bo1, br1, sem0, sem1):
    cid = lax.axis_index("c")
    sid = lax.axis_index("s")
    wid = sid * NC + cid
    base = wid * BPW

    pltpu.sync_copy(s_hbm.at[pl.ds(base, BPW)], sidx_v)
    pltpu.sync_copy(r_hbm.at[pl.ds(base, BPW)], ridx_v)
    pltpu.sync_copy(o_hbm.at[pl.ds(base, BPW)], oidx_v)

    bufs = [(bs0, bo0, br0, sem0), (bs1, bo1, br1, sem1)]

    def start(c):
        bs, bo, br, sem = bufs[c % 2]
        off = c * CHUNK
        hs = pltpu.async_copy(ent_hbm.at[sidx_v.at[pl.ds(off, CHUNK)]], bs, sem)
        ho = pltpu.async_copy(ent_hbm.at[oidx_v.at[pl.ds(off, CHUNK)]], bo, sem)
        hr = pltpu.async_copy(rel_hbm.at[ridx_v.at[pl.ds(off, CHUNK)]], br, sem)
        return (hs, ho, hr)

    iota = lax.iota(jnp.int32, L)
    mask0 = iota == 0
    idx_re = [iota * 2 + 2 * L * j for j in range(EMB // L)]
    idx_im = [iota * 2 + (2 * L * j + 1) for j in range(EMB // L)]

    pending = {0: start(0)}
    for c in range(NCHUNK):
        for h in pending.pop(c):
            h.wait()
        if c + 1 < NCHUNK:
            pending[c + 1] = start(c + 1)
        bs, bo, br, _ = bufs[c % 2]
        out_off = c * CHUNK

        def one_row(rr, bs, bo, br):
            rs = jnp.full((L,), rr, jnp.int32)
            acc = jnp.zeros((L,), jnp.float32)
            for j in range(EMB // L):
                sre = plsc.load_gather(bs, [rs, idx_re[j]])
                sim = plsc.load_gather(bs, [rs, idx_im[j]])
                rre = plsc.load_gather(br, [rs, idx_re[j]])
                rim = plsc.load_gather(br, [rs, idx_im[j]])
                ore = plsc.load_gather(bo, [rs, idx_re[j]])
                oim = plsc.load_gather(bo, [rs, idx_im[j]])
                pre = sre * rre - sim * rim
                pim = sre * rim + sim * rre
                mp = pre * pre + pim * pim
                mo = ore * ore + oim * oim
                cross = pre * ore + pim * oim
                acc = acc + cross * jnp.minimum(_rsqrt3(mp * mo), 1e27)
            tot = jnp.sum(acc)
            ssq = 2.0 * EMB - (tot + tot)
            plsc.store_scatter(out_v, [rs + out_off],
                               jnp.full((L,), ssq, jnp.float32), mask=mask0)

        def row_body(rp, carry, bs=bs, bo=bo, br=br):
            one_row(rp * 2, bs, bo, br)
            one_row(rp * 2 + 1, bs, bo, br)
            return carry

        lax.fori_loop(0, CHUNK // 2, row_body, 0)

    for k in range(BPW // L):
        x = jnp.maximum(out_v[pl.ds(k * L, L)], 0.0)
        y = _rsqrt3(x)
        y = y * (1.5 - 0.5 * x * y * y)
        out_v[pl.ds(k * L, L)] = x * y

    pltpu.sync_copy(out_v, out_hbm.at[pl.ds(base, BPW)])


@jax.jit
def _run(s_idx, r_idx, o_idx, ent_weight, rel_weight):
    mesh = plsc.VectorSubcoreMesh(core_axis_name="c", subcore_axis_name="s")
    f = functools.partial(
        pl.kernel,
        mesh=mesh,
        compiler_params=pltpu.CompilerParams(use_tc_tiling_on_sc=True,
                                             needs_layout_passes=False),
        out_type=jax.ShapeDtypeStruct((B,), jnp.float32),
        scratch_types=[
            pltpu.VMEM((BPW,), jnp.int32),
            pltpu.VMEM((BPW,), jnp.int32),
            pltpu.VMEM((BPW,), jnp.int32),
            pltpu.VMEM((BPW,), jnp.float32),
            pltpu.VMEM((CHUNK, ROW), jnp.float32),
            pltpu.VMEM((CHUNK, ROW), jnp.float32),
            pltpu.VMEM((CHUNK, ROW), jnp.float32),
            pltpu.VMEM((CHUNK, ROW), jnp.float32),
            pltpu.VMEM((CHUNK, ROW), jnp.float32),
            pltpu.VMEM((CHUNK, ROW), jnp.float32),
            pltpu.SemaphoreType.DMA,
            pltpu.SemaphoreType.DMA,
        ],
    )(_body)
    return f(s_idx, r_idx, o_idx, ent_weight, rel_weight)


def kernel(s_idx, r_idx, o_idx, ent_weight, rel_weight):
    return _run(s_idx, r_idx, o_idx, ent_weight, rel_weight)
